# baseline (speedup 1.0000x reference)
import jax
import jax.numpy as jnp
from jax import lax
from jax.experimental import pallas as pl
from jax.experimental.pallas import tpu as pltpu
from jax.experimental.pallas import tpu_sc as plsc

N_SEQ_LEN = 20
NUM_EMBEDDINGS = 100000
EMBEDDING_DIM = 32
BATCH = 4096

_LANES = 16
_NW = 32
_BLK = BATCH // _NW


def _body(idx_hbm, tab_hbm, out_hbm, cols_v, rows_v, isem, sem):
    nc = 2
    wid = lax.axis_index("s") * nc + lax.axis_index("c")
    b0 = wid * _BLK

    pltpu.async_copy(
        idx_hbm.at[pl.ds(0, N_SEQ_LEN), pl.ds(b0, _BLK)], cols_v, isem
    ).wait()

    copies = [
        pltpu.async_copy(tab_hbm.at[d].at[cols_v.at[d]], rows_v.at[d], sem)
        for d in range(N_SEQ_LEN)
    ]
    for d, cp in enumerate(copies):
        cp.wait()
        pltpu.sync_copy(rows_v.at[d], out_hbm.at[pl.ds(b0, _BLK), d])


@jax.jit
def kernel(x, tables):
    idx = x.astype(jnp.int32).T

    mesh = plsc.VectorSubcoreMesh(core_axis_name="c", subcore_axis_name="s")
    run = pl.kernel(
        _body,
        mesh=mesh,
        compiler_params=pltpu.CompilerParams(use_tc_tiling_on_sc=False),
        out_type=jax.ShapeDtypeStruct(
            (BATCH, N_SEQ_LEN, EMBEDDING_DIM), jnp.float32
        ),
        scratch_types=[
            pltpu.VMEM((N_SEQ_LEN, _BLK), jnp.int32),
            pltpu.VMEM((N_SEQ_LEN, _BLK, EMBEDDING_DIM), jnp.float32),
            pltpu.SemaphoreType.DMA,
            pltpu.SemaphoreType.DMA,
        ],
    )
    return run(idx, tables)


# speedup vs baseline: 1.8355x; 1.8355x over previous
import jax
import jax.numpy as jnp
from jax import lax
from jax.experimental import pallas as pl
from jax.experimental.pallas import tpu as pltpu
from jax.experimental.pallas import tpu_sc as plsc

N_SEQ_LEN = 20
NUM_EMBEDDINGS = 100000
EMBEDDING_DIM = 32
BATCH = 4096

_CHUNK = 128
_NCHUNK = BATCH // _CHUNK


def _body(idx_hbm, tab_hbm, out_hbm, idx_v, row_v, isem, gsem):
    nc = 2
    e = lax.axis_index("s") * nc + lax.axis_index("c")

    def task(d, carry):
        row = d * EMBEDDING_DIM + e
        pltpu.async_copy(idx_hbm.at[d], idx_v, isem).wait()
        copies = [
            pltpu.async_copy(
                tab_hbm.at[row].at[idx_v.at[r]], row_v.at[r], gsem
            )
            for r in range(_NCHUNK)
        ]
        for cp in copies:
            cp.wait()
        pltpu.async_copy(row_v, out_hbm.at[row], isem).wait()
        return carry

    lax.fori_loop(0, N_SEQ_LEN, task, 0)


@jax.jit
def kernel(x, tables):
    idx = x.astype(jnp.int32).T.reshape(N_SEQ_LEN, _NCHUNK, _CHUNK)
    tab = jnp.transpose(tables, (0, 2, 1)).reshape(
        N_SEQ_LEN * EMBEDDING_DIM, NUM_EMBEDDINGS
    )

    mesh = plsc.VectorSubcoreMesh(core_axis_name="c", subcore_axis_name="s")
    run = pl.kernel(
        _body,
        mesh=mesh,
        compiler_params=pltpu.CompilerParams(use_tc_tiling_on_sc=False),
        out_type=jax.ShapeDtypeStruct(
            (N_SEQ_LEN * EMBEDDING_DIM, _NCHUNK, _CHUNK), jnp.float32
        ),
        scratch_types=[
            pltpu.VMEM((_NCHUNK, _CHUNK), jnp.int32),
            pltpu.VMEM((_NCHUNK, _CHUNK), jnp.float32),
            pltpu.SemaphoreType.DMA,
            pltpu.SemaphoreType.DMA,
        ],
    )
    out = run(idx, tab)
    out = out.reshape(N_SEQ_LEN, EMBEDDING_DIM, BATCH)
    return jnp.transpose(out, (2, 0, 1))
